# baseline (speedup 1.0000x reference)
import functools

import jax
import jax.numpy as jnp
from jax import lax
from jax.experimental import pallas as pl
from jax.experimental.pallas import tpu as pltpu
from jax.experimental.pallas import tpu_sc as plsc

T, ALPHA, BETA = 20, 0.9, 0.8
B, K, IN = 16, 16, 10000
D0, D1, D2, D3 = 5000, 2500, 1250, 625
P0, P1, P2, P3 = 5120, 2560, 1280, 768
NW = 32
N0, N1, N2, N3 = P0 // NW, P1 // NW, P2 // NW, P3 // NW
C0, C1, C2, C3 = N0 * K // 64, N1 * K // 64, N2 * K // 64, N3 * K // 64


def _mesh():
    return plsc.VectorSubcoreMesh(core_axis_name="c", subcore_axis_name="s")


_CP = pltpu.CompilerParams(use_tc_tiling_on_sc=False)


def _wid():
    return lax.axis_index("c") * 16 + lax.axis_index("s")


_GDN = lax.GatherDimensionNumbers(
    offset_dims=(), collapsed_slice_dims=(0,), start_index_map=(0,))


def _lane(wv, k):
    return lax.gather(wv, jnp.full((B, 1), k, jnp.int32), _GDN, (1,),
                      mode=lax.GatherScatterMode.PROMISE_IN_BOUNDS)


def _wsum(wv, xg_at, init):
    parts = [init, None, None, None]
    for k in range(K):
        t = _lane(wv, k) * xg_at(k)
        p = k % 4
        parts[p] = t if parts[p] is None else parts[p] + t
    return (parts[0] + parts[1]) + (parts[2] + parts[3])


def _spiking_kernel(N, C, stride_out):
    GSRC = 64
    GN = GSRC // K
    NB = 4
    G = N * K // GSRC

    def body(tbl_h, idx_h, wp_h, b_h, th_h, out_h, *scr):
        wp_v, b_v, th_v, idx_v = scr[0], scr[1], scr[2], scr[3]
        xg = scr[4:4 + NB]
        hst_v = scr[4 + NB]
        sem = scr[5 + NB:5 + 2 * NB]
        w = _wid()
        pltpu.sync_copy(wp_h.at[pl.ds(w * N, N)], wp_v)
        pltpu.sync_copy(b_h.at[pl.ds(w * N, N)], b_v)
        pltpu.sync_copy(th_h.at[pl.ds(w * N, N)], th_v)
        pltpu.sync_copy(idx_h.at[w], idx_v)

        def fire(g):
            pltpu.async_copy(tbl_h.at[idx_v.at[g]], xg[g % NB], sem[g % NB])

        def drain(g):
            pltpu.make_async_copy(
                tbl_h.at[idx_v.at[g]], xg[g % NB], sem[g % NB]).wait()

        for g in range(NB - 1):
            fire(g)
        z = jnp.zeros((B,), jnp.float32)
        for g in range(G):
            if g + NB - 1 < G:
                fire(g + NB - 1)
            drain(g)
            xgc = xg[g % NB]
            wvs = [wp_v[g * GN + jj] for jj in range(GN)]
            bs = [b_v[g * GN + jj] for jj in range(GN)]
            ths = [th_v[g * GN + jj] for jj in range(GN)]

            @pl.loop(0, T, init_carry=(z,) * (2 * GN))
            def _(t, carry, _x=xgc, _wvs=wvs, _bs=bs, _ths=ths):
                out = []
                for jj in range(GN):
                    syn0, mem0 = carry[jj], carry[GN + jj]
                    acc = _wsum(_wvs[jj], lambda k, _jj=jj: _x[_jj * K + k, t],
                                _bs[jj])
                    th = _ths[jj]
                    reset = jnp.where(mem0 - th > 0, th, 0.0)
                    syn = ALPHA * syn0 + acc
                    mem = BETA * mem0 + syn - reset
                    out.append((syn, mem))
                    if stride_out is not None:
                        hst_v[jj * T + t] = jnp.where(mem - th > 0, 1.0, 0.0)
                return tuple(s for s, _ in out) + tuple(m for _, m in out)

            fin = _
            if stride_out is not None:
                pltpu.sync_copy(
                    hst_v, out_h.at[pl.ds((w * N + g * GN) * T, GN * T)])
            else:
                for jj in range(GN):
                    hst_v[g * GN + jj] = fin[GN + jj]

        if stride_out is None:
            pltpu.sync_copy(hst_v, out_h.at[pl.ds(w * N, N)])

    hst_rows = GN * T if stride_out is not None else N
    scratch = (
        [pltpu.VMEM((N, K), jnp.float32),
         pltpu.VMEM((N, B), jnp.float32),
         pltpu.VMEM((N, B), jnp.float32),
         pltpu.VMEM((G, GSRC), jnp.int32)]
        + [pltpu.VMEM((GSRC, T, B), jnp.float32)] * NB
        + [pltpu.VMEM((hst_rows, B), jnp.float32)]
        + [pltpu.SemaphoreType.DMA] * NB
    )
    return body, scratch


def _relu_kernel(N, C):
    def body(tbl_h, idx_h, wp_h, b_h, out_h, xg_v, wp_v, b_v, idx_v, o_v, sem):
        w = _wid()
        pltpu.sync_copy(idx_h.at[w], idx_v)
        handles = [
            pltpu.async_copy(tbl_h.at[idx_v.at[c]],
                             xg_v.at[pl.ds(c * 64, 64)], sem)
            for c in range(C)
        ]
        pltpu.sync_copy(wp_h.at[pl.ds(w * N, N)], wp_v)
        pltpu.sync_copy(b_h.at[pl.ds(w * N, N)], b_v)
        for h in handles:
            h.wait()

        @pl.loop(0, N)
        def _(j):
            acc = _wsum(wp_v[j], lambda k: xg_v[j * K + k], b_v[j])
            o_v[j] = jnp.maximum(acc, 0.0)

        pltpu.sync_copy(o_v, out_h.at[pl.ds(w * N, N)])

    return body


def _fc_kernel(x3_h, fcw_h, fcb_h, out_h, x3_v, fcw_v, acc_v, sem):
    w = _wid()
    G = P3 // B

    @pl.when(w == 0)
    def _():
        pltpu.sync_copy(x3_h, x3_v)
        pltpu.sync_copy(fcw_h, fcw_v)
        pltpu.sync_copy(fcb_h, acc_v)
        for o in range(2):
            @pl.loop(0, G)
            def _(g):
                acc_v[o] = _wsum(fcw_v[o * G + g],
                                 lambda k, _g=g: x3_v[_g * B + k], acc_v[o])
        pltpu.sync_copy(acc_v, out_h)


def _pad_rows(a, P):
    pad = P - a.shape[0]
    if pad == 0:
        return a
    return jnp.concatenate([a, jnp.zeros((pad,) + a.shape[1:], a.dtype)], axis=0)


def _prep(knn, W, bvec, P):
    knnp = _pad_rows(knn.astype(jnp.int32), P)
    Wp = _pad_rows(W, P)
    idx = knnp.reshape(NW, -1, 64)
    bb = jnp.broadcast_to(_pad_rows(bvec.reshape(-1, 1), P), (P, B)).astype(jnp.float32)
    return idx, Wp.astype(jnp.float32), bb


def kernel(input, W0, b0, W1, b1, W2, b2, W3, b3, knn0, knn1, knn2, knn3,
           th0, th1, fcW, fcb):
    f32 = jnp.float32
    xT = input.transpose(2, 1, 0)

    idx0, wb0, b0b = _prep(knn0, W0, b0, P0)
    idx1, wb1, b1b = _prep(knn1, W1, b1, P1)
    idx2, wb2, b2b = _prep(knn2, W2, b2, P2)
    idx3, wb3, b3b = _prep(knn3, W3, b3, P3)
    th0b = jnp.broadcast_to(_pad_rows(th0.reshape(-1, 1), P0), (P0, B)).astype(f32)
    th1b = jnp.broadcast_to(_pad_rows(th1.reshape(-1, 1), P1), (P1, B)).astype(f32)
    fcWb = _pad_rows(fcW.T, P3).T.reshape(2 * P3 // B, B).astype(f32)
    fcbb = jnp.broadcast_to(fcb.reshape(-1, 1), (2, B)).astype(f32)

    bodyA, scrA = _spiking_kernel(N0, C0, P0)
    kA = pl.kernel(bodyA, compiler_params=_CP, mesh=_mesh(),
                   out_type=jax.ShapeDtypeStruct((P0 * T, B), f32),
                   scratch_types=scrA)
    h0 = kA(xT, idx0, wb0, b0b, th0b)

    bodyB, scrB = _spiking_kernel(N1, C1, None)
    kB = pl.kernel(bodyB, compiler_params=_CP, mesh=_mesh(),
                   out_type=jax.ShapeDtypeStruct((P1, B), f32),
                   scratch_types=scrB)
    m1 = kB(h0.reshape(P0, T, B), idx1, wb1, b1b, th1b)

    kC = functools.partial(
        pl.kernel, compiler_params=_CP, out_type=jax.ShapeDtypeStruct((P2, B), f32), mesh=_mesh(),
        scratch_types=[
            pltpu.VMEM((N2 * K, B), f32), pltpu.VMEM((N2, K), f32),
            pltpu.VMEM((N2, B), f32), pltpu.VMEM((C2, 64), jnp.int32),
            pltpu.VMEM((N2, B), f32), pltpu.SemaphoreType.DMA,
        ])(_relu_kernel(N2, C2))
    x2 = kC(m1, idx2, wb2, b2b)

    kD = functools.partial(
        pl.kernel, compiler_params=_CP, out_type=jax.ShapeDtypeStruct((P3, B), f32), mesh=_mesh(),
        scratch_types=[
            pltpu.VMEM((N3 * K, B), f32), pltpu.VMEM((N3, K), f32),
            pltpu.VMEM((N3, B), f32), pltpu.VMEM((C3, 64), jnp.int32),
            pltpu.VMEM((N3, B), f32), pltpu.SemaphoreType.DMA,
        ])(_relu_kernel(N3, C3))
    x3 = kD(x2, idx3, wb3, b3b)

    kE = functools.partial(
        pl.kernel, compiler_params=_CP, out_type=jax.ShapeDtypeStruct((2, B), f32), mesh=_mesh(),
        scratch_types=[
            pltpu.VMEM((P3, B), f32), pltpu.VMEM((2 * P3 // B, B), f32),
            pltpu.VMEM((2, B), f32), pltpu.SemaphoreType.DMA,
        ])(_fc_kernel)
    angle = kE(x3, fcWb, fcbb)

    return angle.T


# speedup vs baseline: 1.1582x; 1.1582x over previous
import functools

import jax
import jax.numpy as jnp
from jax import lax
from jax.experimental import pallas as pl
from jax.experimental.pallas import tpu as pltpu
from jax.experimental.pallas import tpu_sc as plsc

T, ALPHA, BETA = 20, 0.9, 0.8
B, K, IN = 16, 16, 10000
D0, D1, D2, D3 = 5000, 2500, 1250, 625
P0, P1, P2, P3 = 5120, 2560, 1280, 768
NW = 32
N0, N1, N2, N3 = P0 // NW, P1 // NW, P2 // NW, P3 // NW
C0, C1, C2, C3 = N0 * K // 64, N1 * K // 64, N2 * K // 64, N3 * K // 64


def _mesh():
    return plsc.VectorSubcoreMesh(core_axis_name="c", subcore_axis_name="s")


_CP = pltpu.CompilerParams(use_tc_tiling_on_sc=False)


def _wid():
    return lax.axis_index("c") * 16 + lax.axis_index("s")


_GDN = lax.GatherDimensionNumbers(
    offset_dims=(), collapsed_slice_dims=(0,), start_index_map=(0,))


def _lane(wv, k):
    return lax.gather(wv, jnp.full((B, 1), k, jnp.int32), _GDN, (1,),
                      mode=lax.GatherScatterMode.PROMISE_IN_BOUNDS)


def _wsum(wv, xg_at, init):
    parts = [init, None, None, None]
    for k in range(K):
        t = _lane(wv, k) * xg_at(k)
        p = k % 4
        parts[p] = t if parts[p] is None else parts[p] + t
    return (parts[0] + parts[1]) + (parts[2] + parts[3])


GSRC = 64
GN = GSRC // K
NB = 4


def _loader(N, idx_h, wp_h, b_h, th_h, wp_v, b_v, th_v, idx_v):
    w = _wid()
    pltpu.sync_copy(wp_h.at[pl.ds(w * N, N)], wp_v)
    pltpu.sync_copy(b_h.at[pl.ds(w * N, N)], b_v)
    pltpu.sync_copy(th_h.at[pl.ds(w * N, N)], th_v)
    pltpu.sync_copy(idx_h.at[w], idx_v)
    return w


def _ring(tbl_h, idx_v, xg, sem):
    def fire(g):
        pltpu.async_copy(tbl_h.at[idx_v.at[g]], xg[g % NB], sem[g % NB])

    def drain(g):
        pltpu.make_async_copy(
            tbl_h.at[idx_v.at[g]], xg[g % NB], sem[g % NB]).wait()

    return fire, drain


def _spk_scratch(N, G, xg_shape, xg_dtype, out_dtype):
    return (
        [pltpu.VMEM((N, K), jnp.float32),
         pltpu.VMEM((N, B), jnp.float32),
         pltpu.VMEM((N, B), jnp.float32),
         pltpu.VMEM((G, GSRC), jnp.int32)]
        + [pltpu.VMEM(xg_shape, xg_dtype)] * NB
        + [pltpu.VMEM((N, B), out_dtype)]
        + [pltpu.SemaphoreType.DMA] * NB
    )


def _phase_a_kernel(N):
    G = N * K // GSRC

    def body(tbl_h, idx_h, wp_h, b_h, th_h, out_h, *scr):
        wp_v, b_v, th_v, idx_v = scr[0], scr[1], scr[2], scr[3]
        xg = scr[4:4 + NB]
        hw_v = scr[4 + NB]
        sem = scr[5 + NB:5 + 2 * NB]
        w = _loader(N, idx_h, wp_h, b_h, th_h, wp_v, b_v, th_v, idx_v)
        fire, drain = _ring(tbl_h, idx_v, xg, sem)

        for g in range(NB - 1):
            fire(g)
        z = jnp.zeros((B,), jnp.float32)
        zi = jnp.zeros((B,), jnp.int32)
        one = jnp.full((B,), 1, jnp.int32)
        for g in range(G):
            if g + NB - 1 < G:
                fire(g + NB - 1)
            drain(g)
            xgc = xg[g % NB]
            wvs = [wp_v[g * GN + jj] for jj in range(GN)]
            bs = [b_v[g * GN + jj] for jj in range(GN)]
            ths = [th_v[g * GN + jj] for jj in range(GN)]

            @pl.loop(0, T, init_carry=(z,) * (2 * GN) + (zi,) * GN + (one,))
            def _(t, carry, _x=xgc, _wvs=wvs, _bs=bs, _ths=ths):
                m = carry[3 * GN]
                syns, mems, words = [], [], []
                for jj in range(GN):
                    syn0, mem0 = carry[jj], carry[GN + jj]
                    acc = _wsum(_wvs[jj], lambda k, _jj=jj: _x[_jj * K + k, t],
                                _bs[jj])
                    th = _ths[jj]
                    reset = jnp.where(mem0 - th > 0, th, 0.0)
                    syn = ALPHA * syn0 + acc
                    mem = BETA * mem0 + syn - reset
                    syns.append(syn)
                    mems.append(mem)
                    words.append(carry[2 * GN + jj]
                                 | jnp.where(mem - th > 0, m, zi))
                return tuple(syns) + tuple(mems) + tuple(words) + (m + m,)

            fin = _
            for jj in range(GN):
                hw_v[g * GN + jj] = fin[2 * GN + jj]

        pltpu.sync_copy(hw_v, out_h.at[pl.ds(w * N, N)])

    return body, _spk_scratch(N, G, (GSRC, T, B), jnp.float32, jnp.int32)


def _phase_b_kernel(N):
    G = N * K // GSRC

    def body(tbl_h, idx_h, wp_h, b_h, th_h, out_h, *scr):
        wp_v, b_v, th_v, idx_v = scr[0], scr[1], scr[2], scr[3]
        xg = scr[4:4 + NB]
        hst_v = scr[4 + NB]
        sem = scr[5 + NB:5 + 2 * NB]
        w = _loader(N, idx_h, wp_h, b_h, th_h, wp_v, b_v, th_v, idx_v)
        fire, drain = _ring(tbl_h, idx_v, xg, sem)

        for g in range(NB - 1):
            fire(g)
        z = jnp.zeros((B,), jnp.float32)
        zf = jnp.zeros((B,), jnp.float32)
        one = jnp.full((B,), 1, jnp.int32)
        for g in range(G):
            if g + NB - 1 < G:
                fire(g + NB - 1)
            drain(g)
            xgc = xg[g % NB]
            for jj in range(GN):
                j = g * GN + jj
                wv = wp_v[j]
                bb = b_v[j]
                th = th_v[j]
                sws = [xgc[jj * K + k] for k in range(K)]

                @pl.loop(0, T, init_carry=(z, z, one))
                def _(t, carry, _sws=sws, _wv=wv, _bb=bb, _th=th):
                    syn0, mem0, m = carry
                    parts = [_bb, None, None, None]
                    for k in range(K):
                        v = jnp.where((_sws[k] & m) > 0, _lane(_wv, k), zf)
                        p = k % 4
                        parts[p] = v if parts[p] is None else parts[p] + v
                    acc = (parts[0] + parts[1]) + (parts[2] + parts[3])
                    reset = jnp.where(mem0 - _th > 0, _th, 0.0)
                    syn = ALPHA * syn0 + acc
                    mem = BETA * mem0 + syn - reset
                    return (syn, mem, m + m)

                hst_v[j] = _[1]

        pltpu.sync_copy(hst_v, out_h.at[pl.ds(w * N, N)])

    return body, _spk_scratch(N, G, (GSRC, B), jnp.int32, jnp.float32)


def _relu_kernel(N, C):
    def body(tbl_h, idx_h, wp_h, b_h, out_h, xg_v, wp_v, b_v, idx_v, o_v, sem):
        w = _wid()
        pltpu.sync_copy(idx_h.at[w], idx_v)
        handles = [
            pltpu.async_copy(tbl_h.at[idx_v.at[c]],
                             xg_v.at[pl.ds(c * 64, 64)], sem)
            for c in range(C)
        ]
        pltpu.sync_copy(wp_h.at[pl.ds(w * N, N)], wp_v)
        pltpu.sync_copy(b_h.at[pl.ds(w * N, N)], b_v)
        for h in handles:
            h.wait()

        @pl.loop(0, N)
        def _(j):
            acc = _wsum(wp_v[j], lambda k: xg_v[j * K + k], b_v[j])
            o_v[j] = jnp.maximum(acc, 0.0)

        pltpu.sync_copy(o_v, out_h.at[pl.ds(w * N, N)])

    return body


def _fc_kernel(x3_h, fcw_h, fcb_h, out_h, x3_v, fcw_v, acc_v, sem):
    w = _wid()
    G = P3 // B

    @pl.when(w == 0)
    def _():
        pltpu.sync_copy(x3_h, x3_v)
        pltpu.sync_copy(fcw_h, fcw_v)
        pltpu.sync_copy(fcb_h, acc_v)
        for o in range(2):
            @pl.loop(0, G)
            def _(g):
                acc_v[o] = _wsum(fcw_v[o * G + g],
                                 lambda k, _g=g: x3_v[_g * B + k], acc_v[o])
        pltpu.sync_copy(acc_v, out_h)


def _pad_rows(a, P):
    pad = P - a.shape[0]
    if pad == 0:
        return a
    return jnp.concatenate([a, jnp.zeros((pad,) + a.shape[1:], a.dtype)], axis=0)


def _prep(knn, W, bvec, P):
    knnp = _pad_rows(knn.astype(jnp.int32), P)
    Wp = _pad_rows(W, P)
    idx = knnp.reshape(NW, -1, 64)
    bb = jnp.broadcast_to(_pad_rows(bvec.reshape(-1, 1), P), (P, B)).astype(jnp.float32)
    return idx, Wp.astype(jnp.float32), bb


def kernel(input, W0, b0, W1, b1, W2, b2, W3, b3, knn0, knn1, knn2, knn3,
           th0, th1, fcW, fcb):
    f32 = jnp.float32
    xT = input.transpose(2, 1, 0)

    idx0, wb0, b0b = _prep(knn0, W0, b0, P0)
    idx1, wb1, b1b = _prep(knn1, W1, b1, P1)
    idx2, wb2, b2b = _prep(knn2, W2, b2, P2)
    idx3, wb3, b3b = _prep(knn3, W3, b3, P3)
    th0b = jnp.broadcast_to(_pad_rows(th0.reshape(-1, 1), P0), (P0, B)).astype(f32)
    th1b = jnp.broadcast_to(_pad_rows(th1.reshape(-1, 1), P1), (P1, B)).astype(f32)
    fcWb = _pad_rows(fcW.T, P3).T.reshape(2 * P3 // B, B).astype(f32)
    fcbb = jnp.broadcast_to(fcb.reshape(-1, 1), (2, B)).astype(f32)

    bodyA, scrA = _phase_a_kernel(N0)
    kA = pl.kernel(bodyA, compiler_params=_CP, mesh=_mesh(),
                   out_type=jax.ShapeDtypeStruct((P0, B), jnp.int32),
                   scratch_types=scrA)
    h0 = kA(xT, idx0, wb0, b0b, th0b)

    bodyB, scrB = _phase_b_kernel(N1)
    kB = pl.kernel(bodyB, compiler_params=_CP, mesh=_mesh(),
                   out_type=jax.ShapeDtypeStruct((P1, B), f32),
                   scratch_types=scrB)
    m1 = kB(h0, idx1, wb1, b1b, th1b)

    kC = functools.partial(
        pl.kernel, compiler_params=_CP, out_type=jax.ShapeDtypeStruct((P2, B), f32), mesh=_mesh(),
        scratch_types=[
            pltpu.VMEM((N2 * K, B), f32), pltpu.VMEM((N2, K), f32),
            pltpu.VMEM((N2, B), f32), pltpu.VMEM((C2, 64), jnp.int32),
            pltpu.VMEM((N2, B), f32), pltpu.SemaphoreType.DMA,
        ])(_relu_kernel(N2, C2))
    x2 = kC(m1, idx2, wb2, b2b)

    kD = functools.partial(
        pl.kernel, compiler_params=_CP, out_type=jax.ShapeDtypeStruct((P3, B), f32), mesh=_mesh(),
        scratch_types=[
            pltpu.VMEM((N3 * K, B), f32), pltpu.VMEM((N3, K), f32),
            pltpu.VMEM((N3, B), f32), pltpu.VMEM((C3, 64), jnp.int32),
            pltpu.VMEM((N3, B), f32), pltpu.SemaphoreType.DMA,
        ])(_relu_kernel(N3, C3))
    x3 = kD(x2, idx3, wb3, b3b)

    kE = functools.partial(
        pl.kernel, compiler_params=_CP, out_type=jax.ShapeDtypeStruct((2, B), f32), mesh=_mesh(),
        scratch_types=[
            pltpu.VMEM((P3, B), f32), pltpu.VMEM((2 * P3 // B, B), f32),
            pltpu.VMEM((2, B), f32), pltpu.SemaphoreType.DMA,
        ])(_fc_kernel)
    angle = kE(x3, fcWb, fcbb)

    return angle.T
